# baseline (speedup 1.0000x reference)
import functools

import jax
import jax.numpy as jnp
from jax import lax
from jax.experimental import pallas as pl
from jax.experimental.pallas import tpu as pltpu
from jax.experimental.pallas import tpu_sc as plsc

B = 1024
D = 512
N = 100000
K = 16
W = 2000
NB = N // W

NW = 32
QW = B // NW
CH = D // 16


def _topk_body(hc_ref, db_ref, ti_ref, mu_ref, r1_ref, q_ref, e_ref):
    j = pl.program_id(0)

    @pl.when(j == 0)
    def _init():
        ti_ref[...] = jnp.zeros((B, K), dtype=jnp.int32)

    h = hc_ref[...]
    hn = h * lax.rsqrt(jnp.maximum(jnp.sum(h * h, axis=1, keepdims=True), 1e-24))
    db = db_ref[...]
    inv = lax.rsqrt(jnp.maximum(jnp.sum(db * db, axis=1), 1e-24))
    s = lax.dot_general(hn.astype(jnp.bfloat16), db.astype(jnp.bfloat16),
                        (((1,), (1,)), ((), ())),
                        preferred_element_type=jnp.float32)
    s = s * inv[None, :]

    IDM = 131071
    colid = lax.broadcasted_iota(jnp.int32, (B, W), 1)
    ubits = lax.bitcast_convert_type(s + 3.0, jnp.int32)
    key = (((ubits >> 9) & jnp.int32(0x3FFF)) << 17) | (IDM - j * W - colid)

    lane = lax.broadcasted_iota(jnp.int32, (B, K), 1)
    tk0 = ti_ref[...]

    NSL = (W + 127) // 128

    def slice_t(t):
        if (t + 1) * 128 <= W:
            return key[:, t * 128:(t + 1) * 128]
        part = key[:, t * 128:W]
        return jnp.concatenate(
            [part, jnp.zeros((B, (t + 1) * 128 - W), jnp.int32)], axis=1)

    def build(flo):
        leaves = []
        for t in range(NSL):
            st = slice_t(t)
            if flo is not None:
                st = jnp.where(st < flo, st, 0)
            leaves.append(st)
        his = leaves
        los = [jnp.zeros_like(leaves[0])] * len(leaves)
        while len(his) > 1:
            nh, nl = [], []
            for i in range(0, len(his) - 1, 2):
                h1, h2 = his[i], his[i + 1]
                l1, l2 = los[i], los[i + 1]
                nh.append(jnp.maximum(h1, h2))
                nl.append(jnp.maximum(jnp.minimum(h1, h2),
                                      jnp.maximum(l1, l2)))
            if len(his) % 2:
                nh.append(his[-1])
                nl.append(los[-1])
            his, los = nh, nl
        return his[0], los[0]

    def inner_body(c):
        _, tk = c
        r1 = r1_ref[...]
        q = q_ref[...]
        need = None
        for _ in range(4):
            vk = jnp.max(r1, axis=1, keepdims=True)
            need = vk > tk[:, K - 1:K]
            mask = r1 == vk
            r1 = jnp.where(mask, jnp.maximum(q, 0), r1)
            q = jnp.where(mask, -vk, q)
            rank = jnp.sum((tk > vk).astype(jnp.int32), axis=1, keepdims=True)
            sk = pltpu.roll(tk, shift=1, axis=1)
            nk = jnp.where(lane < rank, tk, jnp.where(lane == rank, vk, sk))
            tk = jnp.where(need, nk, tk)
        r1_ref[...] = r1
        q_ref[...] = q
        return jnp.any(need), tk

    def outer_body(c):
        _, tk = c
        go0 = jnp.any(jnp.max(r1_ref[...], axis=1, keepdims=True)
                      > tk[:, K - 1:K])
        _, tk = lax.while_loop(lambda cc: cc[0], inner_body, (go0, tk))
        q = q_ref[...]
        refneed = jnp.any((r1_ref[...] == 0) & (e_ref[...] == 0)
                          & ((0 - q) > tk[:, K - 1:K]))

        @pl.when(refneed)
        def _refill():
            flo = jnp.where(q < 0, 0 - q, jnp.int32(0x7FFFFFFF))
            r1, r2 = build(flo)
            r1_ref[...] = r1
            q_ref[...] = jnp.where(q < 0, q, r2)
            e_ref[...] = (r1 == 0).astype(jnp.int32)

        return refneed, tk

    r1b, qb = build(None)
    r1_ref[...] = r1b
    q_ref[...] = qb
    e_ref[...] = jnp.zeros((B, 128), jnp.int32)

    _, tkf = lax.while_loop(lambda c: c[0], outer_body,
                            (jnp.bool_(True), tk0))

    @pl.when(j == NB - 1)
    def _fin():
        tvf = lax.bitcast_convert_type(
            jnp.int32(0x40000000) | (((tkf >> 17) & jnp.int32(0x3FFF)) << 9),
            jnp.float32) - 3.0
        m = jnp.max(tvf, axis=1, keepdims=True)
        e = jnp.exp(tvf - m)
        mu_ref[...] = e / jnp.sum(e, axis=1, keepdims=True)
        ti_ref[...] = IDM - (tkf & jnp.int32(IDM))

    @pl.when(j != NB - 1)
    def _mid():
        ti_ref[...] = tkf


_topk = pl.pallas_call(
    _topk_body,
    grid=(NB,),
    in_specs=[
        pl.BlockSpec((B, D), lambda j: (0, 0)),
        pl.BlockSpec((W, D), lambda j: (j, 0)),
    ],
    out_specs=[
        pl.BlockSpec((B, K), lambda j: (0, 0)),
        pl.BlockSpec((B, K), lambda j: (0, 0)),
    ],
    out_shape=[
        jax.ShapeDtypeStruct((B, K), jnp.int32),
        jax.ShapeDtypeStruct((B, K), jnp.float32),
    ],
    scratch_shapes=[
        pltpu.VMEM((B, 128), jnp.int32),
        pltpu.VMEM((B, 128), jnp.int32),
        pltpu.VMEM((B, 128), jnp.int32),
    ],
    compiler_params=pltpu.CompilerParams(dimension_semantics=("arbitrary",)),
)


def _gather_blend_body(db_hbm, ti_hbm, mu_hbm, hc_hbm, al_hbm, out_hbm,
                       ti_v, mu_v, hc_v, out_v, rows_v, al_v, sem):
    cid = lax.axis_index("c")
    sid = lax.axis_index("s")
    wid = sid * 2 + cid
    q0 = wid * QW
    pltpu.sync_copy(ti_hbm.at[pl.ds(q0, QW)], ti_v)
    pltpu.sync_copy(mu_hbm.at[pl.ds(q0, QW)], mu_v)
    pltpu.sync_copy(hc_hbm.at[pl.ds(q0, QW)], hc_v)
    pltpu.sync_copy(al_hbm, al_v)
    av = al_v[...]
    a = 1.0 / (1.0 + jnp.exp(-av))
    one_m_a = 1.0 - a

    def qloop(q, carry):
        pltpu.async_copy(db_hbm.at[ti_v.at[q]], rows_v, sem).wait()
        mks = [plsc.load_gather(mu_v.at[q], [jnp.full((16,), k, jnp.int32)])
               for k in range(K)]
        for c in range(CH):
            acc = mks[0] * rows_v[0, pl.ds(c * 16, 16)]
            for k in range(1, K):
                acc = acc + mks[k] * rows_v[k, pl.ds(c * 16, 16)]
            hcc = hc_v[q, pl.ds(c * 16, 16)]
            out_v[q, pl.ds(c * 16, 16)] = a * hcc + one_m_a * acc
        return carry

    lax.fori_loop(0, QW, qloop, 0)
    pltpu.sync_copy(out_v, out_hbm.at[pl.ds(q0, QW)])


@functools.cache
def _make_gather_blend():
    return pl.kernel(
        _gather_blend_body,
        out_type=jax.ShapeDtypeStruct((B, D), jnp.float32),
        mesh=plsc.VectorSubcoreMesh(core_axis_name="c", subcore_axis_name="s"),
        compiler_params=pltpu.CompilerParams(needs_layout_passes=False),
        scratch_types=[
            pltpu.VMEM((QW, K), jnp.int32),
            pltpu.VMEM((QW, K), jnp.float32),
            pltpu.VMEM((QW, D), jnp.float32),
            pltpu.VMEM((QW, D), jnp.float32),
            pltpu.VMEM((K, D), jnp.float32),
            pltpu.VMEM((16,), jnp.float32),
            pltpu.SemaphoreType.DMA,
        ],
    )


def kernel(h_current, history_db, alpha):
    ti, mu = _topk(h_current, history_db)
    al = jnp.broadcast_to(jnp.reshape(alpha, (1,)).astype(jnp.float32), (16,))
    return _make_gather_blend()(history_db, ti, mu, h_current, al)


# speedup vs baseline: 1.1015x; 1.1015x over previous
import functools

import jax
import jax.numpy as jnp
from jax import lax
from jax.experimental import pallas as pl
from jax.experimental.pallas import tpu as pltpu
from jax.experimental.pallas import tpu_sc as plsc

B = 1024
D = 512
N = 100000
K = 16
W = 2000
NB = N // W

NW = 32
QW = B // NW
CH = D // 16


def _topk_body(hc_ref, db_ref, ti_ref, mu_ref, r1_ref, q_ref, e_ref):
    j = pl.program_id(0)

    @pl.when(j == 0)
    def _init():
        ti_ref[...] = jnp.zeros((B, K), dtype=jnp.int32)

    h = hc_ref[...]
    hn = h * lax.rsqrt(jnp.maximum(jnp.sum(h * h, axis=1, keepdims=True), 1e-24))
    db = db_ref[...]
    inv = lax.rsqrt(jnp.maximum(jnp.sum(db * db, axis=1), 1e-24))
    s = lax.dot_general(hn.astype(jnp.bfloat16), db.astype(jnp.bfloat16),
                        (((1,), (1,)), ((), ())),
                        preferred_element_type=jnp.float32)
    s = s * inv[None, :]

    IDM = 131071
    colid = lax.broadcasted_iota(jnp.int32, (B, W), 1)
    ubits = lax.bitcast_convert_type(s + 3.0, jnp.int32)
    key = (((ubits >> 9) & jnp.int32(0x3FFF)) << 17) | (IDM - j * W - colid)

    lane = lax.broadcasted_iota(jnp.int32, (B, K), 1)
    tk0 = ti_ref[...]

    NSL = (W + 127) // 128

    def slice_t(t):
        if (t + 1) * 128 <= W:
            return key[:, t * 128:(t + 1) * 128]
        part = key[:, t * 128:W]
        return jnp.concatenate(
            [part, jnp.zeros((B, (t + 1) * 128 - W), jnp.int32)], axis=1)

    def build(flo):
        leaves = []
        for t in range(NSL):
            st = slice_t(t)
            if flo is not None:
                st = jnp.where(st < flo, st, 0)
            leaves.append(st)
        his = leaves
        los = [jnp.zeros_like(leaves[0])] * len(leaves)
        while len(his) > 1:
            nh, nl = [], []
            for i in range(0, len(his) - 1, 2):
                h1, h2 = his[i], his[i + 1]
                l1, l2 = los[i], los[i + 1]
                nh.append(jnp.maximum(h1, h2))
                nl.append(jnp.maximum(jnp.minimum(h1, h2),
                                      jnp.maximum(l1, l2)))
            if len(his) % 2:
                nh.append(his[-1])
                nl.append(los[-1])
            his, los = nh, nl
        return his[0], los[0]

    def inner_body(c):
        _, tk = c
        r1 = r1_ref[...]
        vk = jnp.max(r1, axis=1, keepdims=True)
        need = vk > tk[:, K - 1:K]
        mask = r1 == vk
        q = q_ref[...]
        r1_ref[...] = jnp.where(mask, jnp.maximum(q, 0), r1)
        q_ref[...] = jnp.where(mask, -vk, q)
        rank = jnp.sum((tk > vk).astype(jnp.int32), axis=1, keepdims=True)
        sk = pltpu.roll(tk, shift=1, axis=1)
        nk = jnp.where(lane < rank, tk, jnp.where(lane == rank, vk, sk))
        tk2 = jnp.where(need, nk, tk)
        return jnp.any(need), tk2

    def outer_body(c):
        _, tk = c
        go0 = jnp.any(jnp.max(r1_ref[...], axis=1, keepdims=True)
                      > tk[:, K - 1:K])
        _, tk = lax.while_loop(lambda cc: cc[0], inner_body, (go0, tk))
        q = q_ref[...]
        refneed = jnp.any((r1_ref[...] == 0) & (e_ref[...] == 0)
                          & ((0 - q) > tk[:, K - 1:K]))

        @pl.when(refneed)
        def _refill():
            flo = jnp.where(q < 0, 0 - q, jnp.int32(0x7FFFFFFF))
            r1, r2 = build(flo)
            r1_ref[...] = r1
            q_ref[...] = jnp.where(q < 0, q, r2)
            e_ref[...] = (r1 == 0).astype(jnp.int32)

        return refneed, tk

    r1b, qb = build(None)
    r1_ref[...] = r1b
    q_ref[...] = qb
    e_ref[...] = jnp.zeros((B, 128), jnp.int32)

    _, tkf = lax.while_loop(lambda c: c[0], outer_body,
                            (jnp.bool_(True), tk0))

    @pl.when(j == NB - 1)
    def _fin():
        tvf = lax.bitcast_convert_type(
            jnp.int32(0x40000000) | (((tkf >> 17) & jnp.int32(0x3FFF)) << 9),
            jnp.float32) - 3.0
        m = jnp.max(tvf, axis=1, keepdims=True)
        e = jnp.exp(tvf - m)
        mu_ref[...] = e / jnp.sum(e, axis=1, keepdims=True)
        ti_ref[...] = IDM - (tkf & jnp.int32(IDM))

    @pl.when(j != NB - 1)
    def _mid():
        ti_ref[...] = tkf


_topk = pl.pallas_call(
    _topk_body,
    grid=(NB,),
    in_specs=[
        pl.BlockSpec((B, D), lambda j: (0, 0)),
        pl.BlockSpec((W, D), lambda j: (j, 0)),
    ],
    out_specs=[
        pl.BlockSpec((B, K), lambda j: (0, 0)),
        pl.BlockSpec((B, K), lambda j: (0, 0)),
    ],
    out_shape=[
        jax.ShapeDtypeStruct((B, K), jnp.int32),
        jax.ShapeDtypeStruct((B, K), jnp.float32),
    ],
    scratch_shapes=[
        pltpu.VMEM((B, 128), jnp.int32),
        pltpu.VMEM((B, 128), jnp.int32),
        pltpu.VMEM((B, 128), jnp.int32),
    ],
    compiler_params=pltpu.CompilerParams(dimension_semantics=("arbitrary",)),
)


def _gather_blend_body(db_hbm, ti_hbm, mu_hbm, hc_hbm, al_hbm, out_hbm,
                       ti_v, mu_v, hc_v, out_v, rows_v, al_v, sem):
    cid = lax.axis_index("c")
    sid = lax.axis_index("s")
    wid = sid * 2 + cid
    q0 = wid * QW
    pltpu.sync_copy(ti_hbm.at[pl.ds(q0, QW)], ti_v)
    pltpu.sync_copy(mu_hbm.at[pl.ds(q0, QW)], mu_v)
    pltpu.sync_copy(hc_hbm.at[pl.ds(q0, QW)], hc_v)
    pltpu.sync_copy(al_hbm, al_v)
    av = al_v[...]
    a = 1.0 / (1.0 + jnp.exp(-av))
    one_m_a = 1.0 - a

    def qloop(q, carry):
        pltpu.async_copy(db_hbm.at[ti_v.at[q]], rows_v, sem).wait()
        mks = [plsc.load_gather(mu_v.at[q], [jnp.full((16,), k, jnp.int32)])
               for k in range(K)]
        for c in range(CH):
            acc = mks[0] * rows_v[0, pl.ds(c * 16, 16)]
            for k in range(1, K):
                acc = acc + mks[k] * rows_v[k, pl.ds(c * 16, 16)]
            hcc = hc_v[q, pl.ds(c * 16, 16)]
            out_v[q, pl.ds(c * 16, 16)] = a * hcc + one_m_a * acc
        return carry

    lax.fori_loop(0, QW, qloop, 0)
    pltpu.sync_copy(out_v, out_hbm.at[pl.ds(q0, QW)])


@functools.cache
def _make_gather_blend():
    return pl.kernel(
        _gather_blend_body,
        out_type=jax.ShapeDtypeStruct((B, D), jnp.float32),
        mesh=plsc.VectorSubcoreMesh(core_axis_name="c", subcore_axis_name="s"),
        compiler_params=pltpu.CompilerParams(needs_layout_passes=False),
        scratch_types=[
            pltpu.VMEM((QW, K), jnp.int32),
            pltpu.VMEM((QW, K), jnp.float32),
            pltpu.VMEM((QW, D), jnp.float32),
            pltpu.VMEM((QW, D), jnp.float32),
            pltpu.VMEM((K, D), jnp.float32),
            pltpu.VMEM((16,), jnp.float32),
            pltpu.SemaphoreType.DMA,
        ],
    )


def kernel(h_current, history_db, alpha):
    ti, mu = _topk(h_current, history_db)
    al = jnp.broadcast_to(jnp.reshape(alpha, (1,)).astype(jnp.float32), (16,))
    return _make_gather_blend()(history_db, ti, mu, h_current, al)
